# Initial kernel scaffold; baseline (speedup 1.0000x reference)
import jax
import jax.numpy as jnp
from jax.experimental import pallas as pl


def kernel(x, code_x, edge_index, batch, c1_W1, c1_b1, c1_g, c1_beta, c1_rm, c1_rv, c1_W2, c1_b2, c2_W1, c2_b1, c2_g, c2_beta, c2_rm, c2_rv, c2_W2, c2_b2, lin1_W, lin1_b, lin2_W, lin2_b, fc1_W, fc1_b, fc2_W, fc2_b, fc3_W, fc3_b, fin_W, fin_b):
    raise NotImplementedError("write your pallas kernel here")


# speedup vs baseline: 6.7069x; 6.7069x over previous
import functools

import jax
import jax.numpy as jnp
from jax import lax
from jax.experimental import pallas as pl
from jax.experimental.pallas import tpu as pltpu
from jax.experimental.pallas import tpu_sc as plsc

N = 10000
E = 320000
G = 64
D = 128

NC = 2
NS = 16
NW = NC * NS
EPW = E // NW
CH = 80
NCHUNK = EPW // CH
RPS = 624
TAIL = N - NS * RPS


def _sc_edge_agg_body(feats, src, dst, out, agg, srcs_v, dsts_v, rows, gsem):
    c = lax.axis_index("c")
    s = lax.axis_index("s")
    wid = s * NC + c

    def zrow(i, _):
        for j in range(D // 16):
            rows[i, pl.ds(j * 16, 16)] = jnp.zeros((16,), jnp.float32)
        return _
    lax.fori_loop(0, CH, zrow, None)
    for r in range(RPS // CH):
        pltpu.sync_copy(rows, agg.at[pl.ds(s * RPS + r * CH, CH)])
    pltpu.sync_copy(rows.at[pl.ds(0, RPS % CH)],
                    agg.at[pl.ds(s * RPS + (RPS // CH) * CH, RPS % CH)])

    @pl.when(s == NS - 1)
    def _():
        pltpu.sync_copy(rows.at[pl.ds(0, TAIL)], agg.at[pl.ds(NS * RPS, TAIL)])

    plsc.subcore_barrier()

    pltpu.sync_copy(src.at[wid], srcs_v)
    pltpu.sync_copy(dst.at[wid], dsts_v)

    def step(t, _):
        pltpu.async_copy(feats.at[srcs_v.at[t]], rows, gsem).wait()
        pltpu.sync_copy(rows, agg.at[dsts_v.at[t]], add=True)
        return _
    lax.fori_loop(0, NCHUNK, step, None)

    plsc.subcore_barrier()
    pltpu.sync_copy(agg.at[pl.ds(s * RPS, RPS)], out.at[c, pl.ds(s * RPS, RPS)])

    @pl.when(s == NS - 1)
    def _():
        pltpu.sync_copy(agg.at[pl.ds(NS * RPS, TAIL)],
                        out.at[c, pl.ds(NS * RPS, TAIL)])


@functools.cache
def _sc_edge_agg():
    return pl.kernel(
        _sc_edge_agg_body,
        out_type=jax.ShapeDtypeStruct((NC, N, D), jnp.float32),
        mesh=plsc.VectorSubcoreMesh(
            core_axis_name="c", subcore_axis_name="s",
            num_cores=NC, num_subcores=NS,
        ),
        scratch_types=[
            pltpu.VMEM_SHARED((N, D), jnp.float32),
            pltpu.VMEM((NCHUNK, CH), jnp.int32),
            pltpu.VMEM((NCHUNK, CH), jnp.int32),
            pltpu.VMEM((CH, D), jnp.float32),
            pltpu.SemaphoreType.DMA,
        ],
    )


RB = 2000
NRB = N // RB


def _mlp_body(x_ref, p0_ref, p1_ref, w1_ref, b1_ref, w2_ref, b2_ref, o_ref):
    h = x_ref[...] + p0_ref[...] + p1_ref[...]
    t = jnp.dot(h, w1_ref[...], preferred_element_type=jnp.float32) + b1_ref[...]
    t = jnp.maximum(t, 0.0)
    o = jnp.dot(t, w2_ref[...], preferred_element_type=jnp.float32) + b2_ref[...]
    o_ref[...] = jnp.maximum(o, 0.0)


_row_spec = pl.BlockSpec((RB, D), lambda i: (i, 0))
_w_spec = pl.BlockSpec((D, D), lambda i: (0, 0))
_b_spec = pl.BlockSpec((1, D), lambda i: (0, 0))

_mlp = pl.pallas_call(
    _mlp_body,
    grid=(NRB,),
    in_specs=[_row_spec, _row_spec, _row_spec, _w_spec, _b_spec, _w_spec, _b_spec],
    out_specs=_row_spec,
    out_shape=jax.ShapeDtypeStruct((N, D), jnp.float32),
)


def _mlp_pool_body(x_ref, p0_ref, p1_ref, w1_ref, b1_ref, w2_ref, b2_ref,
                   batch_ref, o_ref):
    i = pl.program_id(0)
    h = x_ref[...] + p0_ref[...] + p1_ref[...]
    t = jnp.dot(h, w1_ref[...], preferred_element_type=jnp.float32) + b1_ref[...]
    t = jnp.maximum(t, 0.0)
    h2 = jnp.dot(t, w2_ref[...], preferred_element_type=jnp.float32) + b2_ref[...]
    h2 = jnp.maximum(h2, 0.0)
    b = batch_ref[0]
    gid = lax.broadcasted_iota(jnp.int32, (G, 1), 0)
    mask = (b == gid).astype(jnp.float32)
    part = jnp.dot(mask, h2, preferred_element_type=jnp.float32)

    @pl.when(i == 0)
    def _():
        o_ref[...] = jnp.zeros_like(o_ref)

    o_ref[...] += part


_mlp_pool = pl.pallas_call(
    _mlp_pool_body,
    grid=(NRB,),
    in_specs=[
        _row_spec, _row_spec, _row_spec, _w_spec, _b_spec, _w_spec, _b_spec,
        pl.BlockSpec((1, 1, RB), lambda i: (i, 0, 0)),
    ],
    out_specs=pl.BlockSpec((G, D), lambda i: (0, 0)),
    out_shape=jax.ShapeDtypeStruct((G, D), jnp.float32),
    compiler_params=pltpu.CompilerParams(dimension_semantics=("arbitrary",)),
)


def _log_softmax(z):
    m = jnp.max(z, axis=1, keepdims=True)
    e = jnp.exp(z - m)
    return z - m - jnp.log(jnp.sum(e, axis=1, keepdims=True))


def _head_body(pooled_ref, code_ref, lin1W_ref, lin1b_ref, lin2W_ref, lin2b_ref,
               fc1W_ref, fc1b_ref, fc2W_ref, fc2b_ref, fc3W_ref, fc3b_ref,
               finA_ref, finB_ref, finb_ref, o_ref):
    t = jnp.dot(pooled_ref[...], lin1W_ref[...],
                preferred_element_type=jnp.float32) + lin1b_ref[...]
    t = jnp.maximum(t, 0.0)
    te = jnp.dot(t, lin2W_ref[...], preferred_element_type=jnp.float32) + lin2b_ref[...]
    c = jnp.dot(code_ref[...], fc1W_ref[...],
                preferred_element_type=jnp.float32) + fc1b_ref[...]
    c = jnp.maximum(c, 0.0)
    c = jnp.dot(c, fc2W_ref[...], preferred_element_type=jnp.float32) + fc2b_ref[...]
    c = jnp.maximum(c, 0.0)
    z = jnp.dot(c, fc3W_ref[...], preferred_element_type=jnp.float32) + fc3b_ref[...]
    ce = _log_softmax(z)
    f = (jnp.dot(ce, finA_ref[...], preferred_element_type=jnp.float32)
         + jnp.dot(te, finB_ref[...], preferred_element_type=jnp.float32)
         + finb_ref[...])
    o_ref[...] = _log_softmax(f)


_head = pl.pallas_call(
    _head_body,
    out_shape=jax.ShapeDtypeStruct((G, G), jnp.float32),
)


def _fold_bn(W1, b1, g, beta, rm, rv):
    s = g / jnp.sqrt(rv + 1e-5)
    return W1 * s[None, :], (b1 - rm) * s + beta


def kernel(x, code_x, edge_index, batch, c1_W1, c1_b1, c1_g, c1_beta, c1_rm, c1_rv, c1_W2, c1_b2, c2_W1, c2_b1, c2_g, c2_beta, c2_rm, c2_rv, c2_W2, c2_b2, lin1_W, lin1_b, lin2_W, lin2_b, fc1_W, fc1_b, fc2_W, fc2_b, fc3_W, fc3_b, fin_W, fin_b):
    src = edge_index[0].reshape(NW, NCHUNK, CH)
    dst = edge_index[1].reshape(NW, NCHUNK, CH)
    batch3 = batch.reshape(NRB, 1, RB)

    W1a, b1a = _fold_bn(c1_W1, c1_b1, c1_g, c1_beta, c1_rm, c1_rv)
    W1b, b1b = _fold_bn(c2_W1, c2_b1, c2_g, c2_beta, c2_rm, c2_rv)

    parts = _sc_edge_agg()(x, src, dst)
    h1 = _mlp(x, parts[0], parts[1], W1a, b1a.reshape(1, D), c1_W2,
              c1_b2.reshape(1, D))
    parts2 = _sc_edge_agg()(h1, src, dst)
    pooled = _mlp_pool(h1, parts2[0], parts2[1], W1b, b1b.reshape(1, D), c2_W2,
                       c2_b2.reshape(1, D), batch3)
    return _head(pooled, code_x, lin1_W, lin1_b.reshape(1, D), lin2_W,
                 lin2_b.reshape(1, D), fc1_W, fc1_b.reshape(1, D), fc2_W,
                 fc2_b.reshape(1, D), fc3_W, fc3_b.reshape(1, D),
                 fin_W[:D], fin_W[D:], fin_b.reshape(1, G))


# speedup vs baseline: 10.7225x; 1.5987x over previous
import functools

import jax
import jax.numpy as jnp
from jax import lax
from jax.experimental import pallas as pl
from jax.experimental.pallas import tpu as pltpu
from jax.experimental.pallas import tpu_sc as plsc

N = 10000
E = 320000
G = 64
D = 128

NC = 2
NS = 16
NW = NC * NS
EPW = E // NW
CH = 80
NCHUNK = EPW // CH
RPS = 624
TAIL = N - NS * RPS


def _sc_edge_agg_body(feats, src, dst, out, agg, srcs_v, dbuf0, dbuf1,
                      rows0, rows1, gsem0, gsem1, ssem0, ssem1, dsem0, dsem1):
    c = lax.axis_index("c")
    s = lax.axis_index("s")
    wid = s * NC + c
    ebase = wid * EPW
    bufs = ((rows0, dbuf0, gsem0, ssem0, dsem0),
            (rows1, dbuf1, gsem1, ssem1, dsem1))

    def zrow(i, _):
        for j in range(D // 16):
            rows0[i, pl.ds(j * 16, 16)] = jnp.zeros((16,), jnp.float32)
        return _
    lax.fori_loop(0, CH, zrow, None)
    for r in range(RPS // CH):
        pltpu.sync_copy(rows0, agg.at[pl.ds(s * RPS + r * CH, CH)])
    pltpu.sync_copy(rows0.at[pl.ds(0, RPS % CH)],
                    agg.at[pl.ds(s * RPS + (RPS // CH) * CH, RPS % CH)])

    @pl.when(s == NS - 1)
    def _():
        pltpu.sync_copy(rows0.at[pl.ds(0, TAIL)], agg.at[pl.ds(NS * RPS, TAIL)])

    plsc.subcore_barrier()

    pltpu.sync_copy(src.at[wid], srcs_v)

    pltpu.async_copy(feats.at[srcs_v.at[0]], rows0, gsem0)
    pltpu.async_copy(dst.at[pl.ds(ebase, CH)], dbuf0, dsem0)
    pltpu.async_copy(feats.at[srcs_v.at[1]], rows1, gsem1)
    pltpu.async_copy(dst.at[pl.ds(ebase + CH, CH)], dbuf1, dsem1)

    def step(i, _):
        for k in range(2):
            t = 2 * i + k
            rows, dbuf, gsem, ssem, dsem = bufs[k]

            @pl.when(t < NCHUNK)
            def _():
                pltpu.make_async_copy(feats.at[srcs_v.at[t]], rows, gsem).wait()
                pltpu.make_async_copy(dst.at[pl.ds(ebase, CH)], dbuf,
                                      dsem).wait()
                pltpu.async_copy(rows, agg.at[dbuf], ssem, add=True).wait()

                @pl.when(t + 2 < NCHUNK)
                def _():
                    pltpu.async_copy(feats.at[srcs_v.at[t + 2]], rows, gsem)
                    pltpu.async_copy(dst.at[pl.ds(ebase + (t + 2) * CH, CH)],
                                     dbuf, dsem)
        return _
    lax.fori_loop(0, (NCHUNK + 1) // 2, step, None)

    plsc.subcore_barrier()
    pltpu.sync_copy(agg.at[pl.ds(s * RPS, RPS)], out.at[c, pl.ds(s * RPS, RPS)])

    @pl.when(s == NS - 1)
    def _():
        pltpu.sync_copy(agg.at[pl.ds(NS * RPS, TAIL)],
                        out.at[c, pl.ds(NS * RPS, TAIL)])


@functools.cache
def _sc_edge_agg():
    return pl.kernel(
        _sc_edge_agg_body,
        out_type=jax.ShapeDtypeStruct((NC, N, D), jnp.float32),
        mesh=plsc.VectorSubcoreMesh(
            core_axis_name="c", subcore_axis_name="s",
            num_cores=NC, num_subcores=NS,
        ),
        scratch_types=[
            pltpu.VMEM_SHARED((N, D), jnp.float32),
            pltpu.VMEM((NCHUNK, CH), jnp.int32),
            pltpu.VMEM((CH,), jnp.int32),
            pltpu.VMEM((CH,), jnp.int32),
            pltpu.VMEM((CH, D), jnp.float32),
            pltpu.VMEM((CH, D), jnp.float32),
            pltpu.SemaphoreType.DMA,
            pltpu.SemaphoreType.DMA,
            pltpu.SemaphoreType.DMA,
            pltpu.SemaphoreType.DMA,
            pltpu.SemaphoreType.DMA,
            pltpu.SemaphoreType.DMA,
        ],
    )


RB = 2000
NRB = N // RB


def _mlp_body(x_ref, p0_ref, p1_ref, w1_ref, b1_ref, w2_ref, b2_ref, o_ref):
    h = x_ref[...] + p0_ref[...] + p1_ref[...]
    t = jnp.dot(h, w1_ref[...], preferred_element_type=jnp.float32) + b1_ref[...]
    t = jnp.maximum(t, 0.0)
    o = jnp.dot(t, w2_ref[...], preferred_element_type=jnp.float32) + b2_ref[...]
    o_ref[...] = jnp.maximum(o, 0.0)


_row_spec = pl.BlockSpec((RB, D), lambda i: (i, 0))
_w_spec = pl.BlockSpec((D, D), lambda i: (0, 0))
_b_spec = pl.BlockSpec((1, D), lambda i: (0, 0))

_mlp = pl.pallas_call(
    _mlp_body,
    grid=(NRB,),
    in_specs=[_row_spec, _row_spec, _row_spec, _w_spec, _b_spec, _w_spec, _b_spec],
    out_specs=_row_spec,
    out_shape=jax.ShapeDtypeStruct((N, D), jnp.float32),
)


def _mlp_pool_body(x_ref, p0_ref, p1_ref, w1_ref, b1_ref, w2_ref, b2_ref,
                   batch_ref, o_ref):
    i = pl.program_id(0)
    h = x_ref[...] + p0_ref[...] + p1_ref[...]
    t = jnp.dot(h, w1_ref[...], preferred_element_type=jnp.float32) + b1_ref[...]
    t = jnp.maximum(t, 0.0)
    h2 = jnp.dot(t, w2_ref[...], preferred_element_type=jnp.float32) + b2_ref[...]
    h2 = jnp.maximum(h2, 0.0)
    b = batch_ref[0]
    gid = lax.broadcasted_iota(jnp.int32, (G, 1), 0)
    mask = (b == gid).astype(jnp.float32)
    part = jnp.dot(mask, h2, preferred_element_type=jnp.float32)

    @pl.when(i == 0)
    def _():
        o_ref[...] = jnp.zeros_like(o_ref)

    o_ref[...] += part


_mlp_pool = pl.pallas_call(
    _mlp_pool_body,
    grid=(NRB,),
    in_specs=[
        _row_spec, _row_spec, _row_spec, _w_spec, _b_spec, _w_spec, _b_spec,
        pl.BlockSpec((1, 1, RB), lambda i: (i, 0, 0)),
    ],
    out_specs=pl.BlockSpec((G, D), lambda i: (0, 0)),
    out_shape=jax.ShapeDtypeStruct((G, D), jnp.float32),
    compiler_params=pltpu.CompilerParams(dimension_semantics=("arbitrary",)),
)


def _log_softmax(z):
    m = jnp.max(z, axis=1, keepdims=True)
    e = jnp.exp(z - m)
    return z - m - jnp.log(jnp.sum(e, axis=1, keepdims=True))


def _head_body(pooled_ref, code_ref, lin1W_ref, lin1b_ref, lin2W_ref, lin2b_ref,
               fc1W_ref, fc1b_ref, fc2W_ref, fc2b_ref, fc3W_ref, fc3b_ref,
               finA_ref, finB_ref, finb_ref, o_ref):
    t = jnp.dot(pooled_ref[...], lin1W_ref[...],
                preferred_element_type=jnp.float32) + lin1b_ref[...]
    t = jnp.maximum(t, 0.0)
    te = jnp.dot(t, lin2W_ref[...], preferred_element_type=jnp.float32) + lin2b_ref[...]
    c = jnp.dot(code_ref[...], fc1W_ref[...],
                preferred_element_type=jnp.float32) + fc1b_ref[...]
    c = jnp.maximum(c, 0.0)
    c = jnp.dot(c, fc2W_ref[...], preferred_element_type=jnp.float32) + fc2b_ref[...]
    c = jnp.maximum(c, 0.0)
    z = jnp.dot(c, fc3W_ref[...], preferred_element_type=jnp.float32) + fc3b_ref[...]
    ce = _log_softmax(z)
    f = (jnp.dot(ce, finA_ref[...], preferred_element_type=jnp.float32)
         + jnp.dot(te, finB_ref[...], preferred_element_type=jnp.float32)
         + finb_ref[...])
    o_ref[...] = _log_softmax(f)


_head = pl.pallas_call(
    _head_body,
    out_shape=jax.ShapeDtypeStruct((G, G), jnp.float32),
)


def _fold_bn(W1, b1, g, beta, rm, rv):
    s = g / jnp.sqrt(rv + 1e-5)
    return W1 * s[None, :], (b1 - rm) * s + beta


def kernel(x, code_x, edge_index, batch, c1_W1, c1_b1, c1_g, c1_beta, c1_rm, c1_rv, c1_W2, c1_b2, c2_W1, c2_b1, c2_g, c2_beta, c2_rm, c2_rv, c2_W2, c2_b2, lin1_W, lin1_b, lin2_W, lin2_b, fc1_W, fc1_b, fc2_W, fc2_b, fc3_W, fc3_b, fin_W, fin_b):
    src = edge_index[0].reshape(NW, NCHUNK, CH)
    dst = edge_index[1]
    batch3 = batch.reshape(NRB, 1, RB)

    W1a, b1a = _fold_bn(c1_W1, c1_b1, c1_g, c1_beta, c1_rm, c1_rv)
    W1b, b1b = _fold_bn(c2_W1, c2_b1, c2_g, c2_beta, c2_rm, c2_rv)

    parts = _sc_edge_agg()(x, src, dst)
    h1 = _mlp(x, parts[0], parts[1], W1a, b1a.reshape(1, D), c1_W2,
              c1_b2.reshape(1, D))
    parts2 = _sc_edge_agg()(h1, src, dst)
    pooled = _mlp_pool(h1, parts2[0], parts2[1], W1b, b1b.reshape(1, D), c2_W2,
                       c2_b2.reshape(1, D), batch3)
    return _head(pooled, code_x, lin1_W, lin1_b.reshape(1, D), lin2_W,
                 lin2_b.reshape(1, D), fc1_W, fc1_b.reshape(1, D), fc2_W,
                 fc2_b.reshape(1, D), fc3_W, fc3_b.reshape(1, D),
                 fin_W[:D], fin_W[D:], fin_b.reshape(1, G))
